# baseline (speedup 1.0000x reference)
import functools

import jax
import jax.numpy as jnp
from jax import lax
from jax.experimental import pallas as pl
from jax.experimental.pallas import tpu as pltpu
from jax.experimental.pallas import tpu_sc as plsc

N = 50000
E = 800000
DIN = 640
H = 128
G = 64
NC, NS, L = 2, 16, 16
NW = NC * NS
EPAD = 819200
PAD = EPAD - E
EPW = EPAD // NW
CAP = EPW + 128
HALF = 25008
ACC2 = HALF + 8
HW = H // NC
RB = 1000
NBLK = N // RB
DEGN = 51200

NSLOT = 3
DG = 2
DS = 3
PF = 3
NIDX = 8


def _mesh():
    return plsc.VectorSubcoreMesh(
        core_axis_name="c", subcore_axis_name="s", num_cores=NC,
        num_subcores=NS)


def _deg(dstr):
    @functools.partial(
        pl.kernel,
        out_type=jax.ShapeDtypeStruct((NC, DEGN), jnp.float32),
        mesh=_mesh(),
        scratch_types=[
            pltpu.VMEM_SHARED((DEGN,), jnp.float32),
            pltpu.VMEM((3200,), jnp.float32),
            pltpu.VMEM((128,), jnp.float32),
            pltpu.VMEM((8, 128), jnp.int32),
        ],
        compiler_params=pltpu.CompilerParams(use_tc_tiling_on_sc=False),
    )
    def deg_kernel(dst_hbm, degp_hbm, degacc, zb, ob, db):
        c = lax.axis_index("c")
        s = lax.axis_index("s")

        def fill(j, _):
            zb[pl.ds(j * L, L)] = jnp.zeros((L,), jnp.float32)
            return 0
        lax.fori_loop(0, 3200 // L, fill, 0)

        def fill1(j, _):
            ob[pl.ds(j * L, L)] = jnp.ones((L,), jnp.float32)
            return 0
        lax.fori_loop(0, 128 // L, fill1, 0)

        pltpu.sync_copy(zb, degacc.at[pl.ds(s * 3200, 3200)])
        plsc.subcore_barrier()

        base = c * 3200 + s * 200

        def grp(g, _):
            r0 = base + g * 8
            pltpu.sync_copy(dst_hbm.at[pl.ds(r0, 8), :], db)
            for j in range(8):
                pltpu.sync_copy(ob, degacc.at[db.at[j]], add=True)
            return 0
        lax.fori_loop(0, 25, grp, 0)

        plsc.subcore_barrier()
        pltpu.sync_copy(degacc.at[pl.ds(s * 3200, 3200)],
                        degp_hbm.at[c, pl.ds(s * 3200, 3200)])

    return deg_kernel(dstr)


def _part(srcp, dstp):
    @functools.partial(
        pl.kernel,
        out_type=(
            jax.ShapeDtypeStruct((2, NW, CAP), jnp.int32),
            jax.ShapeDtypeStruct((2, NW, CAP), jnp.int32),
            jax.ShapeDtypeStruct((2, NW, 16), jnp.int32),
        ),
        mesh=_mesh(),
        scratch_types=[
            pltpu.VMEM((1024,), jnp.int32),
            pltpu.VMEM((1024,), jnp.int32),
            pltpu.VMEM((CAP,), jnp.int32),
            pltpu.VMEM((CAP,), jnp.int32),
            pltpu.VMEM((CAP,), jnp.int32),
            pltpu.VMEM((CAP,), jnp.int32),
            pltpu.VMEM((16,), jnp.int32),
        ],
        compiler_params=pltpu.CompilerParams(use_tc_tiling_on_sc=False, needs_layout_passes=False),
    )
    def part_kernel(src_hbm, dst_hbm, sparts, dparts, cnts,
                    ins, ind, s0, d0, s1, d1, cv):
        c = lax.axis_index("c")
        s = lax.axis_index("s")
        wid = c * NS + s
        e0 = wid * EPW

        def chunk(ch, carry):
            o0, o1 = carry
            pltpu.sync_copy(src_hbm.at[pl.ds(e0 + ch * 1024, 1024)], ins)
            pltpu.sync_copy(dst_hbm.at[pl.ds(e0 + ch * 1024, 1024)], ind)

            def vec(v, carry2):
                p0, p1 = carry2
                sv = ins[pl.ds(v * L, L)]
                dv = ind[pl.ds(v * L, L)]
                m = dv < HALF
                nm = jnp.logical_not(m)
                m01 = m.astype(jnp.int32)
                cum = plsc.cumsum(m01)
                ex0 = cum - m01
                lane = lax.iota(jnp.int32, L)
                ex1 = lane - ex0 - m01
                plsc.store_scatter(s0, [p0 + ex0], sv, mask=m)
                plsc.store_scatter(d0, [p0 + ex0], dv, mask=m)
                plsc.store_scatter(s1, [p1 + ex1], sv, mask=nm)
                plsc.store_scatter(d1, [p1 + ex1], dv - HALF, mask=nm)
                pc = cum[L - 1]
                return (p0 + pc, p1 + (L - pc))
            return lax.fori_loop(0, 1024 // L, vec, (o0, o1))
        o0, o1 = lax.fori_loop(0, EPW // 1024, chunk,
                               (jnp.int32(0), jnp.int32(0)))

        tsrc = jnp.zeros((L,), jnp.int32)
        tdst = HALF + (lax.iota(jnp.int32, L) & 7)
        for r in range(8):
            s0[pl.ds(o0 + r * L, L)] = tsrc
            d0[pl.ds(o0 + r * L, L)] = tdst
            s1[pl.ds(o1 + r * L, L)] = tsrc
            d1[pl.ds(o1 + r * L, L)] = tdst
        pc0 = lax.bitwise_and(o0 + 127, jnp.int32(~127))
        pc1 = lax.bitwise_and(o1 + 127, jnp.int32(~127))

        pltpu.sync_copy(s0, sparts.at[0, wid])
        pltpu.sync_copy(d0, dparts.at[0, wid])
        pltpu.sync_copy(s1, sparts.at[1, wid])
        pltpu.sync_copy(d1, dparts.at[1, wid])
        cv[...] = jnp.ones((16,), jnp.int32) * pc0
        pltpu.sync_copy(cv, cnts.at[0, wid])
        cv[...] = jnp.ones((16,), jnp.int32) * pc1
        pltpu.sync_copy(cv, cnts.at[1, wid])

    return part_kernel(srcp, dstp)


def _agg(ypA, ypB, sparts, dparts, cnts, zeros):
    @functools.partial(
        pl.kernel,
        out_type=(
            jax.ShapeDtypeStruct((N, HW), jnp.float32),
            jax.ShapeDtypeStruct((N, HW), jnp.float32),
        ),
        mesh=_mesh(),
        scratch_types=[
            pltpu.VMEM_SHARED((ACC2, HW), jnp.float32),
            pltpu.VMEM((NIDX * 128,), jnp.int32),
            pltpu.VMEM((NIDX, 1, 128), jnp.int32),
            pltpu.VMEM((NSLOT * 128, HW), jnp.float32),
            pltpu.VMEM((16,), jnp.int32),
            pltpu.SemaphoreType.DMA,
            pltpu.SemaphoreType.DMA,
            pltpu.SemaphoreType.DMA,
        ],
        compiler_params=pltpu.CompilerParams(use_tc_tiling_on_sc=False),
    )
    def agg_kernel(ypa_hbm, ypb_hbm, src_hbm, dst_hbm, cnt_hbm, zero_hbm,
                   outa_hbm, outb_hbm, acc, sb, db, gb, cv,
                   sem_i, sem_g, sem_s):
        c = lax.axis_index("c")
        s = lax.axis_index("s")

        def run_bucket(yp_hbm, h, b):
            pltpu.sync_copy(cnt_hbm.at[h, b], cv)
            ng = lax.div(cv[...][0], jnp.int32(128))

            def idx_load(g, slot):
                pltpu.async_copy(src_hbm.at[h, b, pl.ds(g * 128, 128)],
                                 sb.at[pl.ds(slot * 128, 128)], sem_i)
                pltpu.async_copy(dst_hbm.at[h, b, pl.ds(g * 128, 128)],
                                 db.at[slot, 0], sem_i)

            def idx_wait():
                pltpu.make_async_copy(src_hbm.at[0, 0, pl.ds(0, 128)],
                                      sb.at[pl.ds(0, 128)], sem_i).wait()
                pltpu.make_async_copy(dst_hbm.at[0, 0, pl.ds(0, 128)],
                                      db.at[0, 0], sem_i).wait()

            def gather_issue(islot, bslot):
                pltpu.async_copy(yp_hbm.at[sb.at[pl.ds(islot * 128, 128)]],
                                 gb.at[pl.ds(bslot * 128, 128), :], sem_g)

            def gather_wait():
                pltpu.make_async_copy(yp_hbm.at[sb.at[pl.ds(0, 128)]],
                                      gb.at[pl.ds(0, 128), :], sem_g).wait()

            def scatter_issue(islot, bslot):
                pltpu.async_copy(gb.at[pl.ds(bslot * 128, 128), :],
                                 acc.at[db.at[islot, 0]], sem_s, add=True)

            def scatter_wait():
                pltpu.make_async_copy(gb.at[pl.ds(0, 128), :],
                                      acc.at[db.at[0, 0]], sem_s).wait()

            for p in range(PF):
                @pl.when(p < ng)
                def _():
                    idx_load(p, p)

            def grp(g, _):
                @pl.when(g >= DS)
                def _():
                    scatter_wait()

                @pl.when(g < ng - PF)
                def _():
                    idx_load(g + PF, lax.rem(g + PF, NIDX))

                idx_wait()
                gather_issue(lax.rem(g, NIDX), lax.rem(g, NSLOT))

                @pl.when(g >= DG)
                def _():
                    gather_wait()
                    scatter_issue(lax.rem(g - DG, NIDX),
                                  lax.rem(g - DG, NSLOT))
                return 0
            lax.fori_loop(0, ng, grp, 0)

            gpend = jnp.minimum(jnp.int32(DG), ng)
            gbase = jnp.maximum(ng - DG, 0)
            for r in range(DG):
                @pl.when(r < gpend)
                def _():
                    gather_wait()
                    scatter_issue(lax.rem(gbase + r, NIDX),
                                  lax.rem(gbase + r, NSLOT))
            spend = jnp.minimum(jnp.int32(DS), ng)
            for r in range(DS):
                @pl.when(r < spend)
                def _():
                    scatter_wait()

        def writeout(out_hbm, h, n15):
            @pl.when(s < NS - 1)
            def _():
                pltpu.sync_copy(acc.at[pl.ds(s * 1568, 1568), :],
                                out_hbm.at[pl.ds(h * HALF + s * 1568, 1568), :])
            @pl.when(s == NS - 1)
            def _():
                pltpu.sync_copy(acc.at[pl.ds(23520, n15), :],
                                out_hbm.at[pl.ds(h * HALF + 23520, n15), :])

        for h in range(2):
            @pl.when(s < NS - 1)
            def _():
                pltpu.sync_copy(zero_hbm, acc.at[pl.ds(s * 1568, 1568), :])
            @pl.when(s == NS - 1)
            def _():
                pltpu.sync_copy(zero_hbm.at[pl.ds(0, 1496), :],
                                acc.at[pl.ds(23520, 1496), :])
            plsc.subcore_barrier()

            for q in range(2):
                b = s * 2 + q

                @pl.when(c == 0)
                def _():
                    run_bucket(ypa_hbm, h, b)

                @pl.when(c == 1)
                def _():
                    run_bucket(ypb_hbm, h, b)
            plsc.subcore_barrier()

            n15 = 1488 if h == 0 else 1472

            @pl.when(c == 0)
            def _():
                writeout(outa_hbm, h, n15)

            @pl.when(c == 1)
            def _():
                writeout(outb_hbm, h, n15)
            plsc.subcore_barrier()

    return agg_kernel(ypA, ypB, sparts, dparts, cnts, zeros)


def _dis_of(degp_ref):
    deg = degp_ref[:, 0] + degp_ref[:, 1] + 1.0
    return lax.rsqrt(deg)


def _mm1(emb, degp, W1):
    def body(emb_ref, degp_ref, w_ref, o_ref):
        dis = _dis_of(degp_ref)
        acc = jnp.dot(emb_ref[...], w_ref[...],
                      preferred_element_type=jnp.float32)
        o_ref[...] = acc * dis[:, None]

    return pl.pallas_call(
        body,
        grid=(NBLK,),
        in_specs=[
            pl.BlockSpec((RB, DIN), lambda i: (i, 0)),
            pl.BlockSpec((RB, NC), lambda i: (i, 0)),
            pl.BlockSpec((DIN, H), lambda i: (0, 0)),
        ],
        out_specs=pl.BlockSpec((RB, H), lambda i: (i, 0)),
        out_shape=jax.ShapeDtypeStruct((N, H), jnp.float32),
    )(emb, degp, W1)


def _mm2(SA, SB, yp, degp, W3, b1r):
    def body(sa_ref, sb_ref, yp_ref, degp_ref, w_ref, b_ref, o_ref):
        dis = _dis_of(degp_ref)
        s_full = jnp.concatenate([sa_ref[...], sb_ref[...]], axis=1)
        h = jnp.maximum((s_full + yp_ref[...]) * dis[:, None] + b_ref[...],
                        0.0)
        acc = jnp.dot(h, w_ref[...], preferred_element_type=jnp.float32)
        o_ref[...] = acc * dis[:, None]

    return pl.pallas_call(
        body,
        grid=(NBLK,),
        in_specs=[
            pl.BlockSpec((RB, HW), lambda i: (i, 0)),
            pl.BlockSpec((RB, HW), lambda i: (i, 0)),
            pl.BlockSpec((RB, H), lambda i: (i, 0)),
            pl.BlockSpec((RB, NC), lambda i: (i, 0)),
            pl.BlockSpec((H, H), lambda i: (0, 0)),
            pl.BlockSpec((1, H), lambda i: (0, 0)),
        ],
        out_specs=pl.BlockSpec((RB, H), lambda i: (i, 0)),
        out_shape=jax.ShapeDtypeStruct((N, H), jnp.float32),
    )(SA, SB, yp, degp, W3, b1r)


def _pool(SA, SB, zp, degp, b3r, batch2d):
    def body(sa_ref, sb_ref, zp_ref, degp_ref, b_ref, bat_ref, o_ref,
             acc, cnt):
        i = pl.program_id(0)

        @pl.when(i == 0)
        def _():
            acc[...] = jnp.zeros_like(acc)
            cnt[...] = jnp.zeros_like(cnt)

        dis = _dis_of(degp_ref)
        s_full = jnp.concatenate([sa_ref[...], sb_ref[...]], axis=1)
        x2 = (s_full + zp_ref[...]) * dis[:, None] + b_ref[...]
        gi = lax.broadcasted_iota(jnp.int32, (RB, G), 1)
        oh_t = (bat_ref[...] == gi).astype(jnp.float32)
        dn = (((0,), (0,)), ((), ()))
        acc[...] += lax.dot_general(oh_t, x2, dn,
                                    preferred_element_type=jnp.float32)
        cnt[...] += lax.dot_general(oh_t, jnp.ones_like(x2), dn,
                                    preferred_element_type=jnp.float32)

        @pl.when(i == NBLK - 1)
        def _():
            o_ref[...] = acc[...] / jnp.maximum(cnt[...], 1.0)

    return pl.pallas_call(
        body,
        grid=(NBLK,),
        in_specs=[
            pl.BlockSpec((RB, HW), lambda i: (i, 0)),
            pl.BlockSpec((RB, HW), lambda i: (i, 0)),
            pl.BlockSpec((RB, H), lambda i: (i, 0)),
            pl.BlockSpec((RB, NC), lambda i: (i, 0)),
            pl.BlockSpec((1, H), lambda i: (0, 0)),
            pl.BlockSpec((RB, 1), lambda i: (i, 0)),
        ],
        out_specs=pl.BlockSpec((G, H), lambda i: (0, 0)),
        out_shape=jax.ShapeDtypeStruct((G, H), jnp.float32),
        scratch_shapes=[
            pltpu.VMEM((G, H), jnp.float32),
            pltpu.VMEM((G, H), jnp.float32),
        ],
    )(SA, SB, zp, degp, b3r, batch2d)


def kernel(emb, edge_index, batch, W1, b1, W3, b3):
    src = edge_index[0].astype(jnp.int32)
    dst = edge_index[1].astype(jnp.int32)
    ar = jnp.arange(PAD, dtype=jnp.int32)
    srcp = jnp.concatenate([src, (ar * 13) % N])
    dstp = jnp.concatenate([dst, 2 * HALF + (ar & 7)])
    dstr = dstp.reshape(EPAD // 128, 128)
    zeros = jnp.zeros((1568, HW), jnp.float32)

    degp = _deg(dstr).T
    sparts, dparts, cnts = _part(srcp, dstp)
    yp = _mm1(emb, degp, W1)
    S1A, S1B = _agg(yp[:, :HW], yp[:, HW:], sparts, dparts, cnts, zeros)
    zp = _mm2(S1A, S1B, yp, degp, W3, b1.reshape(1, H))
    S2A, S2B = _agg(zp[:, :HW], zp[:, HW:], sparts, dparts, cnts, zeros)
    return _pool(S2A, S2B, zp, degp, b3.reshape(1, H),
                 batch.astype(jnp.int32).reshape(N, 1))


# speedup vs baseline: 1.1305x; 1.1305x over previous
import functools

import jax
import jax.numpy as jnp
from jax import lax
from jax.experimental import pallas as pl
from jax.experimental.pallas import tpu as pltpu
from jax.experimental.pallas import tpu_sc as plsc

N = 50000
E = 800000
DIN = 640
H = 128
G = 64
NC, NS, L = 2, 16, 16
EPAD = 819200
ROWS = EPAD // 128
PAD = EPAD - E
NCHUNK = 4
CW = H // NCHUNK
ACC_ROWS = N + 16
RB = 1000
NBLK = N // RB
DEGN = 51200


def _mesh():
    return plsc.VectorSubcoreMesh(
        core_axis_name="c", subcore_axis_name="s", num_cores=NC,
        num_subcores=NS)


def _deg(dstr):
    @functools.partial(
        pl.kernel,
        out_type=jax.ShapeDtypeStruct((NC, DEGN), jnp.float32),
        mesh=_mesh(),
        scratch_types=[
            pltpu.VMEM_SHARED((DEGN,), jnp.float32),
            pltpu.VMEM((3200,), jnp.float32),
            pltpu.VMEM((128,), jnp.float32),
            pltpu.VMEM((8, 128), jnp.int32),
        ],
        compiler_params=pltpu.CompilerParams(use_tc_tiling_on_sc=False),
    )
    def deg_kernel(dst_hbm, degp_hbm, degacc, zb, ob, db):
        c = lax.axis_index("c")
        s = lax.axis_index("s")

        def fill(j, _):
            zb[pl.ds(j * L, L)] = jnp.zeros((L,), jnp.float32)
            return 0
        lax.fori_loop(0, 3200 // L, fill, 0)

        def fill1(j, _):
            ob[pl.ds(j * L, L)] = jnp.ones((L,), jnp.float32)
            return 0
        lax.fori_loop(0, 128 // L, fill1, 0)

        pltpu.sync_copy(zb, degacc.at[pl.ds(s * 3200, 3200)])
        plsc.subcore_barrier()

        base = c * (ROWS // NC) + s * 200

        def grp(g, _):
            r0 = base + g * 8
            pltpu.sync_copy(dst_hbm.at[pl.ds(r0, 8), :], db)
            for j in range(8):
                pltpu.sync_copy(ob, degacc.at[db.at[j]], add=True)
            return 0
        lax.fori_loop(0, 25, grp, 0)

        plsc.subcore_barrier()
        pltpu.sync_copy(degacc.at[pl.ds(s * 3200, 3200)],
                        degp_hbm.at[c, pl.ds(s * 3200, 3200)])

    return deg_kernel(dstr)


GR = 1
GE = GR * 128
NG = (ROWS // NS) // GR
NSLOT = 6
DG = 4
DS = 5
PF = 3
NIDX = 10


def _agg(ypflat, src4f, dstr, zeros):
    @functools.partial(
        pl.kernel,
        out_type=jax.ShapeDtypeStruct((N, NCHUNK, CW), jnp.float32),
        mesh=_mesh(),
        scratch_types=[
            pltpu.VMEM_SHARED((ACC_ROWS, CW), jnp.float32),
            pltpu.VMEM((NIDX * GE,), jnp.int32),
            pltpu.VMEM((NIDX, GR, 128), jnp.int32),
            pltpu.VMEM((NSLOT * GE, CW), jnp.float32),
            pltpu.SemaphoreType.DMA,
            pltpu.SemaphoreType.DMA,
            pltpu.SemaphoreType.DMA,
        ],
        compiler_params=pltpu.CompilerParams(use_tc_tiling_on_sc=False),
    )
    def agg_kernel(yp_hbm, src_hbm, dst_hbm, zero_hbm, out_hbm,
                   acc, sb, db, gb, sem_i, sem_g, sem_s):
        c = lax.axis_index("c")
        s = lax.axis_index("s")

        def idx_load(chunk, g, slot):
            e0 = (s * (ROWS // NS) + g * GR) * 128
            pltpu.async_copy(src_hbm.at[chunk, pl.ds(e0, GE)],
                             sb.at[pl.ds(slot * GE, GE)], sem_i)
            pltpu.async_copy(
                dst_hbm.at[pl.ds(s * (ROWS // NS) + g * GR, GR), :],
                db.at[slot], sem_i)

        def idx_wait():
            pltpu.make_async_copy(src_hbm.at[0, pl.ds(0, GE)],
                                  sb.at[pl.ds(0, GE)], sem_i).wait()
            pltpu.make_async_copy(dst_hbm.at[pl.ds(0, GR), :],
                                  db.at[0], sem_i).wait()

        def gather_issue(islot, bslot):
            pltpu.async_copy(yp_hbm.at[sb.at[pl.ds(islot * GE, GE)]],
                             gb.at[pl.ds(bslot * GE, GE), :], sem_g)

        def gather_wait():
            pltpu.make_async_copy(yp_hbm.at[sb.at[pl.ds(0, GE)]],
                                  gb.at[pl.ds(0, GE), :], sem_g).wait()

        def scatter_issue(islot, bslot):
            for j in range(GR):
                pltpu.async_copy(gb.at[pl.ds(bslot * GE + j * 128, 128), :],
                                 acc.at[db.at[islot, j]], sem_s, add=True)

        def scatter_wait():
            for j in range(GR):
                pltpu.make_async_copy(gb.at[pl.ds(j * 128, 128), :],
                                      acc.at[db.at[0, 0]], sem_s).wait()

        def per_chunk(k, _):
            chunk = c * 2 + k
            @pl.when(s < NS - 1)
            def _():
                pltpu.sync_copy(zero_hbm,
                                acc.at[pl.ds(s * 3200, 3200), :])
            @pl.when(s == NS - 1)
            def _():
                pltpu.sync_copy(zero_hbm.at[pl.ds(0, 2016), :],
                                acc.at[pl.ds(48000, 2016), :])
            plsc.subcore_barrier()

            for p in range(PF):
                idx_load(chunk, p, p)

            def grp(g, _):
                @pl.when(g >= DS)
                def _():
                    scatter_wait()

                @pl.when(g < NG - PF)
                def _():
                    idx_load(chunk, g + PF, lax.rem(g + PF, NIDX))

                idx_wait()
                gather_issue(lax.rem(g, NIDX), lax.rem(g, NSLOT))

                @pl.when(g >= DG)
                def _():
                    gather_wait()
                    scatter_issue(lax.rem(g - DG, NIDX), lax.rem(g - DG, NSLOT))
                return 0
            lax.fori_loop(0, NG, grp, 0, unroll=False)

            for r in range(DG):
                gather_wait()
                scatter_issue(lax.rem(NG - DG + r, NIDX),
                              lax.rem(NG - DG + r, NSLOT))
            for r in range(DS):
                scatter_wait()

            plsc.subcore_barrier()
            @pl.when(s < NS - 1)
            def _():
                pltpu.sync_copy(acc.at[pl.ds(s * 3200, 3200), :],
                                out_hbm.at[pl.ds(s * 3200, 3200), chunk, :])
            @pl.when(s == NS - 1)
            def _():
                pltpu.sync_copy(acc.at[pl.ds(48000, 2000), :],
                                out_hbm.at[pl.ds(48000, 2000), chunk, :])
            plsc.subcore_barrier()
            return 0
        lax.fori_loop(0, 2, per_chunk, 0)

    return agg_kernel(ypflat, src4f, dstr, zeros)


def _dis_of(degp_ref):
    deg = degp_ref[:, 0] + degp_ref[:, 1] + 1.0
    return lax.rsqrt(deg)


def _mm1(emb, degp, W1):
    def body(emb_ref, degp_ref, w_ref, o_ref):
        dis = _dis_of(degp_ref)
        acc = jnp.dot(emb_ref[...].astype(jnp.bfloat16),
                      w_ref[...].astype(jnp.bfloat16),
                      preferred_element_type=jnp.float32)
        o_ref[...] = acc * dis[:, None]

    return pl.pallas_call(
        body,
        grid=(NBLK,),
        in_specs=[
            pl.BlockSpec((RB, DIN), lambda i: (i, 0)),
            pl.BlockSpec((RB, NC), lambda i: (i, 0)),
            pl.BlockSpec((DIN, H), lambda i: (0, 0)),
        ],
        out_specs=pl.BlockSpec((RB, H), lambda i: (i, 0)),
        out_shape=jax.ShapeDtypeStruct((N, H), jnp.float32),
    )(emb, degp, W1)


def _mm2(S1, yp, degp, W3, b1r):
    def body(s_ref, yp_ref, degp_ref, w_ref, b_ref, o_ref):
        dis = _dis_of(degp_ref)
        h = jnp.maximum((s_ref[...] + yp_ref[...]) * dis[:, None] + b_ref[...],
                        0.0)
        acc = jnp.dot(h, w_ref[...], preferred_element_type=jnp.float32)
        o_ref[...] = acc * dis[:, None]

    return pl.pallas_call(
        body,
        grid=(NBLK,),
        in_specs=[
            pl.BlockSpec((RB, H), lambda i: (i, 0)),
            pl.BlockSpec((RB, H), lambda i: (i, 0)),
            pl.BlockSpec((RB, NC), lambda i: (i, 0)),
            pl.BlockSpec((H, H), lambda i: (0, 0)),
            pl.BlockSpec((1, H), lambda i: (0, 0)),
        ],
        out_specs=pl.BlockSpec((RB, H), lambda i: (i, 0)),
        out_shape=jax.ShapeDtypeStruct((N, H), jnp.float32),
    )(S1, yp, degp, W3, b1r)


def _pool(S2, zp, degp, b3r, batch2d):
    def body(s_ref, zp_ref, degp_ref, b_ref, bat_ref, o_ref, acc, cnt):
        i = pl.program_id(0)

        @pl.when(i == 0)
        def _():
            acc[...] = jnp.zeros_like(acc)
            cnt[...] = jnp.zeros_like(cnt)

        dis = _dis_of(degp_ref)
        x2 = (s_ref[...] + zp_ref[...]) * dis[:, None] + b_ref[...]
        gi = lax.broadcasted_iota(jnp.int32, (RB, G), 1)
        oh_t = (bat_ref[...] == gi).astype(jnp.float32)
        dn = (((0,), (0,)), ((), ()))
        acc[...] += lax.dot_general(oh_t, x2, dn,
                                    preferred_element_type=jnp.float32)
        cnt[...] += lax.dot_general(oh_t, jnp.ones_like(x2), dn,
                                    preferred_element_type=jnp.float32)

        @pl.when(i == NBLK - 1)
        def _():
            o_ref[...] = acc[...] / jnp.maximum(cnt[...], 1.0)

    return pl.pallas_call(
        body,
        grid=(NBLK,),
        in_specs=[
            pl.BlockSpec((RB, H), lambda i: (i, 0)),
            pl.BlockSpec((RB, H), lambda i: (i, 0)),
            pl.BlockSpec((RB, NC), lambda i: (i, 0)),
            pl.BlockSpec((1, H), lambda i: (0, 0)),
            pl.BlockSpec((RB, 1), lambda i: (i, 0)),
        ],
        out_specs=pl.BlockSpec((G, H), lambda i: (0, 0)),
        out_shape=jax.ShapeDtypeStruct((G, H), jnp.float32),
        scratch_shapes=[
            pltpu.VMEM((G, H), jnp.float32),
            pltpu.VMEM((G, H), jnp.float32),
        ],
    )(S2, zp, degp, b3r, batch2d)


def kernel(emb, edge_index, batch, W1, b1, W3, b3):
    src = edge_index[0].astype(jnp.int32)
    dst = edge_index[1].astype(jnp.int32)
    ar = jnp.arange(PAD, dtype=jnp.int32)
    srcp = jnp.concatenate([src, (ar * 13) % N])
    dstp = jnp.concatenate([dst, N + (ar % 16)])
    src4 = (srcp[None, :] * NCHUNK
            + jnp.arange(NCHUNK, dtype=jnp.int32)[:, None]
            ).reshape(NCHUNK, ROWS, 128)
    dstr = dstp.reshape(ROWS, 128)

    zeros = jnp.zeros((3200, CW), jnp.float32)
    degp = _deg(dstr).T
    yp = _mm1(emb, degp, W1)
    src4f = src4.reshape(NCHUNK, EPAD)
    S1 = _agg(yp.reshape(NCHUNK * N, CW), src4f, dstr, zeros).reshape(N, H)
    zp = _mm2(S1, yp, degp, W3, b1.reshape(1, H))
    S2 = _agg(zp.reshape(NCHUNK * N, CW), src4f, dstr, zeros).reshape(N, H)
    return _pool(S2, zp, degp, b3.reshape(1, H),
                 batch.astype(jnp.int32).reshape(N, 1))


# speedup vs baseline: 1.1306x; 1.0000x over previous
import functools

import jax
import jax.numpy as jnp
from jax import lax
from jax.experimental import pallas as pl
from jax.experimental.pallas import tpu as pltpu
from jax.experimental.pallas import tpu_sc as plsc

N = 50000
E = 800000
DIN = 640
H = 128
G = 64
NC, NS, L = 2, 16, 16
EPAD = 819200
ROWS = EPAD // 128
PAD = EPAD - E
NCHUNK = 4
CW = H // NCHUNK
ACC_ROWS = N + 16
RB = 1000
NBLK = N // RB
DEGN = 51200


def _mesh():
    return plsc.VectorSubcoreMesh(
        core_axis_name="c", subcore_axis_name="s", num_cores=NC,
        num_subcores=NS)


def _deg(dstr):
    @functools.partial(
        pl.kernel,
        out_type=jax.ShapeDtypeStruct((NC, DEGN), jnp.float32),
        mesh=_mesh(),
        scratch_types=[
            pltpu.VMEM_SHARED((DEGN,), jnp.float32),
            pltpu.VMEM((3200,), jnp.float32),
            pltpu.VMEM((128,), jnp.float32),
            pltpu.VMEM((8, 128), jnp.int32),
        ],
        compiler_params=pltpu.CompilerParams(use_tc_tiling_on_sc=False),
    )
    def deg_kernel(dst_hbm, degp_hbm, degacc, zb, ob, db):
        c = lax.axis_index("c")
        s = lax.axis_index("s")

        def fill(j, _):
            zb[pl.ds(j * L, L)] = jnp.zeros((L,), jnp.float32)
            return 0
        lax.fori_loop(0, 3200 // L, fill, 0)

        def fill1(j, _):
            ob[pl.ds(j * L, L)] = jnp.ones((L,), jnp.float32)
            return 0
        lax.fori_loop(0, 128 // L, fill1, 0)

        pltpu.sync_copy(zb, degacc.at[pl.ds(s * 3200, 3200)])
        plsc.subcore_barrier()

        base = c * (ROWS // NC) + s * 200

        def grp(g, _):
            r0 = base + g * 8
            pltpu.sync_copy(dst_hbm.at[pl.ds(r0, 8), :], db)
            for j in range(8):
                pltpu.sync_copy(ob, degacc.at[db.at[j]], add=True)
            return 0
        lax.fori_loop(0, 25, grp, 0)

        plsc.subcore_barrier()
        pltpu.sync_copy(degacc.at[pl.ds(s * 3200, 3200)],
                        degp_hbm.at[c, pl.ds(s * 3200, 3200)])

    return deg_kernel(dstr)


GR = 1
GE = GR * 128
NG = (ROWS // NS) // GR
NSLOT = 6
DG = 4
DS = 5
PF = 3
NIDX = 10


def _agg(ypflat, src4f, dstr, zeros):
    @functools.partial(
        pl.kernel,
        out_type=jax.ShapeDtypeStruct((N, NCHUNK, CW), jnp.float32),
        mesh=_mesh(),
        scratch_types=[
            pltpu.VMEM_SHARED((ACC_ROWS, CW), jnp.float32),
            pltpu.VMEM((NIDX * GE,), jnp.int32),
            pltpu.VMEM((NIDX, GR, 128), jnp.int32),
            pltpu.VMEM((NSLOT * GE, CW), jnp.float32),
            pltpu.SemaphoreType.DMA,
            pltpu.SemaphoreType.DMA,
            pltpu.SemaphoreType.DMA,
        ],
        compiler_params=pltpu.CompilerParams(use_tc_tiling_on_sc=False),
    )
    def agg_kernel(yp_hbm, src_hbm, dst_hbm, zero_hbm, out_hbm,
                   acc, sb, db, gb, sem_i, sem_g, sem_s):
        c = lax.axis_index("c")
        s = lax.axis_index("s")

        def idx_load(chunk, g, slot):
            e0 = (s * (ROWS // NS) + g * GR) * 128
            pltpu.async_copy(src_hbm.at[chunk, pl.ds(e0, GE)],
                             sb.at[pl.ds(slot * GE, GE)], sem_i)
            pltpu.async_copy(
                dst_hbm.at[pl.ds(s * (ROWS // NS) + g * GR, GR), :],
                db.at[slot], sem_i)

        def idx_wait():
            pltpu.make_async_copy(src_hbm.at[0, pl.ds(0, GE)],
                                  sb.at[pl.ds(0, GE)], sem_i).wait()
            pltpu.make_async_copy(dst_hbm.at[pl.ds(0, GR), :],
                                  db.at[0], sem_i).wait()

        def gather_issue(islot, bslot):
            pltpu.async_copy(yp_hbm.at[sb.at[pl.ds(islot * GE, GE)]],
                             gb.at[pl.ds(bslot * GE, GE), :], sem_g)

        def gather_wait():
            pltpu.make_async_copy(yp_hbm.at[sb.at[pl.ds(0, GE)]],
                                  gb.at[pl.ds(0, GE), :], sem_g).wait()

        def scatter_issue(islot, bslot):
            for j in range(GR):
                pltpu.async_copy(gb.at[pl.ds(bslot * GE + j * 128, 128), :],
                                 acc.at[db.at[islot, j]], sem_s, add=True)

        def scatter_wait():
            for j in range(GR):
                pltpu.make_async_copy(gb.at[pl.ds(j * 128, 128), :],
                                      acc.at[db.at[0, 0]], sem_s).wait()

        def per_chunk(k, _):
            chunk = c * 2 + k
            @pl.when(s < NS - 1)
            def _():
                pltpu.sync_copy(zero_hbm,
                                acc.at[pl.ds(s * 3200, 3200), :])
            @pl.when(s == NS - 1)
            def _():
                pltpu.sync_copy(zero_hbm.at[pl.ds(0, 2016), :],
                                acc.at[pl.ds(48000, 2016), :])
            plsc.subcore_barrier()

            for p in range(PF):
                idx_load(chunk, p, p)

            def grp(g, _):
                @pl.when(g >= DS)
                def _():
                    scatter_wait()

                @pl.when(g < NG - PF)
                def _():
                    idx_load(chunk, g + PF, lax.rem(g + PF, NIDX))

                idx_wait()
                gather_issue(lax.rem(g, NIDX), lax.rem(g, NSLOT))

                @pl.when(g >= DG)
                def _():
                    gather_wait()
                    scatter_issue(lax.rem(g - DG, NIDX), lax.rem(g - DG, NSLOT))
                return 0
            lax.fori_loop(0, NG, grp, 0, unroll=False)

            for r in range(DG):
                gather_wait()
                scatter_issue(lax.rem(NG - DG + r, NIDX),
                              lax.rem(NG - DG + r, NSLOT))
            for r in range(DS):
                scatter_wait()

            plsc.subcore_barrier()
            @pl.when(s < NS - 1)
            def _():
                pltpu.sync_copy(acc.at[pl.ds(s * 3200, 3200), :],
                                out_hbm.at[pl.ds(s * 3200, 3200), chunk, :])
            @pl.when(s == NS - 1)
            def _():
                pltpu.sync_copy(acc.at[pl.ds(48000, 2000), :],
                                out_hbm.at[pl.ds(48000, 2000), chunk, :])
            plsc.subcore_barrier()
            return 0
        lax.fori_loop(0, 2, per_chunk, 0)

    return agg_kernel(ypflat, src4f, dstr, zeros)


def _dis_of(degp_ref):
    deg = degp_ref[:, 0] + degp_ref[:, 1] + 1.0
    return lax.rsqrt(deg)


def _mm1(emb, degp, W1):
    def body(emb_ref, degp_ref, w_ref, o_ref):
        dis = _dis_of(degp_ref)
        acc = jnp.dot(emb_ref[...], w_ref[...],
                      preferred_element_type=jnp.float32)
        o_ref[...] = acc * dis[:, None]

    return pl.pallas_call(
        body,
        grid=(NBLK,),
        in_specs=[
            pl.BlockSpec((RB, DIN), lambda i: (i, 0)),
            pl.BlockSpec((RB, NC), lambda i: (i, 0)),
            pl.BlockSpec((DIN, H), lambda i: (0, 0)),
        ],
        out_specs=pl.BlockSpec((RB, H), lambda i: (i, 0)),
        out_shape=jax.ShapeDtypeStruct((N, H), jnp.float32),
    )(emb, degp, W1)


def _mm2(S1, yp, degp, W3, b1r):
    def body(s_ref, yp_ref, degp_ref, w_ref, b_ref, o_ref):
        dis = _dis_of(degp_ref)
        h = jnp.maximum((s_ref[...] + yp_ref[...]) * dis[:, None] + b_ref[...],
                        0.0)
        acc = jnp.dot(h, w_ref[...], preferred_element_type=jnp.float32)
        o_ref[...] = acc * dis[:, None]

    return pl.pallas_call(
        body,
        grid=(NBLK,),
        in_specs=[
            pl.BlockSpec((RB, H), lambda i: (i, 0)),
            pl.BlockSpec((RB, H), lambda i: (i, 0)),
            pl.BlockSpec((RB, NC), lambda i: (i, 0)),
            pl.BlockSpec((H, H), lambda i: (0, 0)),
            pl.BlockSpec((1, H), lambda i: (0, 0)),
        ],
        out_specs=pl.BlockSpec((RB, H), lambda i: (i, 0)),
        out_shape=jax.ShapeDtypeStruct((N, H), jnp.float32),
    )(S1, yp, degp, W3, b1r)


def _pool(S2, zp, degp, b3r, batch2d):
    def body(s_ref, zp_ref, degp_ref, b_ref, bat_ref, o_ref, acc, cnt):
        i = pl.program_id(0)

        @pl.when(i == 0)
        def _():
            acc[...] = jnp.zeros_like(acc)
            cnt[...] = jnp.zeros_like(cnt)

        dis = _dis_of(degp_ref)
        x2 = (s_ref[...] + zp_ref[...]) * dis[:, None] + b_ref[...]
        gi = lax.broadcasted_iota(jnp.int32, (RB, G), 1)
        oh_t = (bat_ref[...] == gi).astype(jnp.float32)
        dn = (((0,), (0,)), ((), ()))
        acc[...] += lax.dot_general(oh_t, x2, dn,
                                    preferred_element_type=jnp.float32)
        cnt[...] += lax.dot_general(oh_t, jnp.ones_like(x2), dn,
                                    preferred_element_type=jnp.float32)

        @pl.when(i == NBLK - 1)
        def _():
            o_ref[...] = acc[...] / jnp.maximum(cnt[...], 1.0)

    return pl.pallas_call(
        body,
        grid=(NBLK,),
        in_specs=[
            pl.BlockSpec((RB, H), lambda i: (i, 0)),
            pl.BlockSpec((RB, H), lambda i: (i, 0)),
            pl.BlockSpec((RB, NC), lambda i: (i, 0)),
            pl.BlockSpec((1, H), lambda i: (0, 0)),
            pl.BlockSpec((RB, 1), lambda i: (i, 0)),
        ],
        out_specs=pl.BlockSpec((G, H), lambda i: (0, 0)),
        out_shape=jax.ShapeDtypeStruct((G, H), jnp.float32),
        scratch_shapes=[
            pltpu.VMEM((G, H), jnp.float32),
            pltpu.VMEM((G, H), jnp.float32),
        ],
    )(S2, zp, degp, b3r, batch2d)


def kernel(emb, edge_index, batch, W1, b1, W3, b3):
    src = edge_index[0].astype(jnp.int32)
    dst = edge_index[1].astype(jnp.int32)
    ar = jnp.arange(PAD, dtype=jnp.int32)
    srcp = jnp.concatenate([src, (ar * 13) % N])
    dstp = jnp.concatenate([dst, N + (ar % 16)])
    src4 = (srcp[None, :] * NCHUNK
            + jnp.arange(NCHUNK, dtype=jnp.int32)[:, None]
            ).reshape(NCHUNK, ROWS, 128)
    dstr = dstp.reshape(ROWS, 128)

    zeros = jnp.zeros((3200, CW), jnp.float32)
    degp = _deg(dstr).T
    yp = _mm1(emb, degp, W1)
    src4f = src4.reshape(NCHUNK, EPAD)
    S1 = _agg(yp.reshape(NCHUNK * N, CW), src4f, dstr, zeros).reshape(N, H)
    zp = _mm2(S1, yp, degp, W3, b1.reshape(1, H))
    S2 = _agg(zp.reshape(NCHUNK * N, CW), src4f, dstr, zeros).reshape(N, H)
    return _pool(S2, zp, degp, b3.reshape(1, H),
                 batch.astype(jnp.int32).reshape(N, 1))
